# baseline (speedup 1.0000x reference)
import functools

import jax
import jax.numpy as jnp
from jax.experimental import pallas as pl
from jax.experimental.pallas import tpu as pltpu

_EPS = 1e-5
_CONTRACT0 = (((0,), (0,)), ((), ()))


def _lshift(a, s):
    if s == 0:
        return a
    pad = jnp.zeros((a.shape[0], abs(s)), a.dtype)
    if s > 0:
        return jnp.concatenate([a[:, s:], pad], axis=1)
    return jnp.concatenate([pad, a[:, :s]], axis=1)


def _stats(y):
    s = jnp.sum(y, axis=1, keepdims=True)
    ss = jnp.sum(y * y, axis=1, keepdims=True)
    return jnp.concatenate([s, ss], axis=1)[None]


def _fold(st_ref, g_ref, be_ref, m):
    tot = jnp.sum(st_ref[...], axis=0)
    mean = tot[:, 0:1] / m
    var = tot[:, 1:2] / m - mean * mean
    sc = jnp.transpose(g_ref[...]) * jax.lax.rsqrt(var + _EPS)
    sh = jnp.transpose(be_ref[...]) - mean * sc
    return sc, sh


def _s1_kernel(x_ref, w_ref, y_ref, xf_ref, st_ref):
    g = x_ref.shape[0]
    xc = jnp.concatenate([x_ref[i] for i in range(g)], axis=1)
    xc = xc.astype(jnp.bfloat16)
    xf_ref[...] = xc
    y = jax.lax.dot_general(w_ref[...], xc, _CONTRACT0,
                            preferred_element_type=jnp.float32)
    y_ref[...] = y.astype(jnp.bfloat16)
    st_ref[...] = _stats(y)


def _s2_kernel(y1_ref, st_ref, g_ref, be_ref, w_ref, y_ref, st2_ref, *, m, width, hw):
    sc, sh = _fold(st_ref, g_ref, be_ref, m)
    c, tl = y1_ref.shape
    lane = jax.lax.broadcasted_iota(jnp.int32, (1, tl), 1)
    col = lane % width
    rowpos = lane % hw
    scb, shb = sc.astype(jnp.bfloat16), sh.astype(jnp.bfloat16)
    zero = jnp.bfloat16(0)
    a = jnp.maximum(y1_ref[...] * scb + shb, zero)
    a_l = jnp.where(col > 0, _lshift(a, -1), zero)
    a_r = jnp.where(col < width - 1, _lshift(a, 1), zero)
    a3 = jnp.concatenate([a_l, a, a_r], axis=0)
    k3 = 3 * c
    y = jax.lax.dot_general(w_ref[k3:2 * k3], a3, _CONTRACT0,
                            preferred_element_type=jnp.float32)
    t_up = jnp.where(rowpos >= width, _lshift(a3, -width), zero)
    y = y + jax.lax.dot_general(w_ref[0:k3], t_up, _CONTRACT0,
                                preferred_element_type=jnp.float32)
    t_dn = jnp.where(rowpos < hw - width, _lshift(a3, width), zero)
    y = y + jax.lax.dot_general(w_ref[2 * k3:3 * k3], t_dn, _CONTRACT0,
                                preferred_element_type=jnp.float32)
    y_ref[...] = y.astype(jnp.bfloat16)
    st2_ref[...] = _stats(y)


def _s3_kernel(y2_ref, st_ref, g_ref, be_ref, w_ref, y_ref, st3_ref, *, m):
    sc, sh = _fold(st_ref, g_ref, be_ref, m)
    scb, shb = sc.astype(jnp.bfloat16), sh.astype(jnp.bfloat16)
    a = jnp.maximum(y2_ref[...] * scb + shb, jnp.bfloat16(0))
    y = jax.lax.dot_general(w_ref[...], a, _CONTRACT0,
                            preferred_element_type=jnp.float32)
    y_ref[...] = y.astype(jnp.bfloat16)
    st3_ref[...] = _stats(y)


def _s4_kernel(y3_ref, st_ref, g_ref, be_ref, x_ref, o_ref, *, m, hw):
    sc, sh = _fold(st_ref, g_ref, be_ref, m)
    o = jnp.maximum(y3_ref[...].astype(jnp.float32) * sc + sh + x_ref[...], 0.0)
    o = o.astype(jnp.bfloat16)
    for i in range(o_ref.shape[0]):
        o_ref[i] = o[:, i * hw:(i + 1) * hw]


def kernel(w1, b1, g1, be1, w2, b2, g2, be2, w3, b3, g3, be3, x):
    N, C, H, W = x.shape
    HW = H * W
    M = N * HW
    f32, bf16 = jnp.float32, jnp.bfloat16
    G = N
    for g in range(1, N + 1):
        if N % g == 0 and (g * HW) % 128 == 0:
            G = g
            break
    nst = N // G
    TL = G * HW

    x3 = x.reshape(N, C, HW)
    w1b = w1.astype(bf16)
    w2b = w2.reshape(9 * C, C).astype(bf16)
    w3b = w3.astype(bf16)
    g1r, be1r = g1.reshape(1, C), be1.reshape(1, C)
    g2r, be2r = g2.reshape(1, C), be2.reshape(1, C)
    g3r, be3r = g3.reshape(1, C), be3.reshape(1, C)

    par = pltpu.CompilerParams(dimension_semantics=("parallel",))
    img = pl.BlockSpec((G, C, HW), lambda n: (n, 0, 0))
    cm = pl.BlockSpec((C, TL), lambda n: (0, n))
    stat_o = pl.BlockSpec((1, C, 2), lambda n: (n, 0, 0))
    stat_i = pl.BlockSpec((nst, C, 2), lambda n: (0, 0, 0))
    row = pl.BlockSpec((1, C), lambda n: (0, 0))

    def mat(shape):
        return pl.BlockSpec(shape, lambda n: (0, 0))

    act_cm = jax.ShapeDtypeStruct((C, M), bf16)
    st_f32 = jax.ShapeDtypeStruct((nst, C, 2), f32)

    y1, xf, st1 = pl.pallas_call(
        _s1_kernel,
        grid=(nst,),
        in_specs=[img, mat((C, C))],
        out_specs=[cm, cm, stat_o],
        out_shape=[act_cm, act_cm, st_f32],
        compiler_params=par,
    )(x3, w1b)

    y2, st2 = pl.pallas_call(
        functools.partial(_s2_kernel, m=M, width=W, hw=HW),
        grid=(nst,),
        in_specs=[cm, stat_i, row, row, mat((9 * C, C))],
        out_specs=[cm, stat_o],
        out_shape=[act_cm, st_f32],
        compiler_params=par,
    )(y1, st1, g1r, be1r, w2b)

    y3, st3 = pl.pallas_call(
        functools.partial(_s3_kernel, m=M),
        grid=(nst,),
        in_specs=[cm, stat_i, row, row, mat((C, C))],
        out_specs=[cm, stat_o],
        out_shape=[act_cm, st_f32],
        compiler_params=par,
    )(y2, st2, g2r, be2r, w3b)

    out = pl.pallas_call(
        functools.partial(_s4_kernel, m=M, hw=HW),
        grid=(nst,),
        in_specs=[cm, stat_i, row, row, cm],
        out_specs=img,
        out_shape=jax.ShapeDtypeStruct((N, C, HW), bf16),
        compiler_params=par,
    )(y3, st3, g3r, be3r, xf)

    return out.reshape(N, C, H, W).astype(f32)


# speedup vs baseline: 1.0978x; 1.0978x over previous
import functools

import jax
import jax.numpy as jnp
from jax.experimental import pallas as pl
from jax.experimental.pallas import tpu as pltpu

_EPS = 1e-5
_CONTRACT0 = (((0,), (0,)), ((), ()))


def _lshift(a, s):
    if s == 0:
        return a
    pad = jnp.zeros((a.shape[0], abs(s)), a.dtype)
    if s > 0:
        return jnp.concatenate([a[:, s:], pad], axis=1)
    return jnp.concatenate([pad, a[:, :s]], axis=1)


def _stats(y):
    s = jnp.sum(y, axis=1, keepdims=True)
    ss = jnp.sum(y * y, axis=1, keepdims=True)
    return jnp.concatenate([s, ss], axis=1)[None]


def _fold(st_ref, g_ref, be_ref, m):
    tot = jnp.sum(st_ref[...], axis=0)
    mean = tot[:, 0:1] / m
    var = tot[:, 1:2] / m - mean * mean
    sc = jnp.transpose(g_ref[...]) * jax.lax.rsqrt(var + _EPS)
    sh = jnp.transpose(be_ref[...]) - mean * sc
    return sc, sh


def _s1_kernel(x_ref, w_ref, y_ref, xf_ref, st_ref):
    g = x_ref.shape[0]
    xc = jnp.concatenate([x_ref[i] for i in range(g)], axis=1)
    xc = xc.astype(jnp.bfloat16)
    xf_ref[...] = xc
    y = jax.lax.dot_general(w_ref[...], xc, _CONTRACT0,
                            preferred_element_type=jnp.float32)
    y_ref[...] = y.astype(jnp.bfloat16)
    st_ref[...] = _stats(y)


def _s2_kernel(y1_ref, st_ref, g_ref, be_ref, w_ref, y_ref, st2_ref, *, m, width, hw):
    sc, sh = _fold(st_ref, g_ref, be_ref, m)
    c, tl = y1_ref.shape
    lane = jax.lax.broadcasted_iota(jnp.int32, (1, tl), 1)
    col = lane % width
    rowpos = lane % hw
    scb, shb = sc.astype(jnp.bfloat16), sh.astype(jnp.bfloat16)
    zero = jnp.bfloat16(0)
    a = jnp.maximum(y1_ref[...] * scb + shb, zero)
    a_l = jnp.where(col > 0, _lshift(a, -1), zero)
    a_r = jnp.where(col < width - 1, _lshift(a, 1), zero)
    a3 = jnp.concatenate([a_l, a, a_r], axis=0)
    k3 = 3 * c
    t_up = jnp.where(rowpos >= width, _lshift(a3, -width), zero)
    y = jax.lax.dot_general(w_ref[0:k3], t_up, _CONTRACT0,
                            preferred_element_type=jnp.float32)
    y = y + jax.lax.dot_general(w_ref[k3:2 * k3], a3, _CONTRACT0,
                                preferred_element_type=jnp.float32)
    t_dn = jnp.where(rowpos < hw - width, _lshift(a3, width), zero)
    y = y + jax.lax.dot_general(w_ref[2 * k3:3 * k3], t_dn, _CONTRACT0,
                                preferred_element_type=jnp.float32)
    y_ref[...] = y.astype(jnp.bfloat16)
    st2_ref[...] = _stats(y)


def _s3_kernel(y2_ref, st_ref, g_ref, be_ref, w_ref, y_ref, st3_ref, *, m):
    sc, sh = _fold(st_ref, g_ref, be_ref, m)
    scb, shb = sc.astype(jnp.bfloat16), sh.astype(jnp.bfloat16)
    a = jnp.maximum(y2_ref[...] * scb + shb, jnp.bfloat16(0))
    y = jax.lax.dot_general(w_ref[...], a, _CONTRACT0,
                            preferred_element_type=jnp.float32)
    y_ref[...] = y.astype(jnp.bfloat16)
    st3_ref[...] = _stats(y)


def _s4_kernel(y3_ref, st_ref, g_ref, be_ref, x_ref, o_ref, *, m, hw):
    sc, sh = _fold(st_ref, g_ref, be_ref, m)
    o = jnp.maximum(y3_ref[...].astype(jnp.float32) * sc + sh + x_ref[...], 0.0)
    o = o.astype(jnp.bfloat16)
    for i in range(o_ref.shape[0]):
        o_ref[i] = o[:, i * hw:(i + 1) * hw]


def kernel(w1, b1, g1, be1, w2, b2, g2, be2, w3, b3, g3, be3, x):
    N, C, H, W = x.shape
    HW = H * W
    M = N * HW
    f32, bf16 = jnp.float32, jnp.bfloat16
    G = N
    for g in range(1, N + 1):
        if N % g == 0 and (g * HW) % 128 == 0:
            G = g
            break
    nst = N // G
    TL = G * HW

    x3 = x.reshape(N, C, HW)
    w1b = w1.astype(bf16)
    w2b = w2.reshape(9 * C, C).astype(bf16)
    w3b = w3.astype(bf16)
    g1r, be1r = g1.reshape(1, C), be1.reshape(1, C)
    g2r, be2r = g2.reshape(1, C), be2.reshape(1, C)
    g3r, be3r = g3.reshape(1, C), be3.reshape(1, C)

    par = pltpu.CompilerParams(dimension_semantics=("parallel",))
    img = pl.BlockSpec((G, C, HW), lambda n: (n, 0, 0))
    cm = pl.BlockSpec((C, TL), lambda n: (0, n))
    stat_o = pl.BlockSpec((1, C, 2), lambda n: (n, 0, 0))
    stat_i = pl.BlockSpec((nst, C, 2), lambda n: (0, 0, 0))
    row = pl.BlockSpec((1, C), lambda n: (0, 0))

    def mat(shape):
        return pl.BlockSpec(shape, lambda n: (0, 0))

    act_cm = jax.ShapeDtypeStruct((C, M), bf16)
    st_f32 = jax.ShapeDtypeStruct((nst, C, 2), f32)

    y1, xf, st1 = pl.pallas_call(
        _s1_kernel,
        grid=(nst,),
        in_specs=[img, mat((C, C))],
        out_specs=[cm, cm, stat_o],
        out_shape=[act_cm, act_cm, st_f32],
        compiler_params=par,
    )(x3, w1b)

    y2, st2 = pl.pallas_call(
        functools.partial(_s2_kernel, m=M, width=W, hw=HW),
        grid=(nst,),
        in_specs=[cm, stat_i, row, row, mat((9 * C, C))],
        out_specs=[cm, stat_o],
        out_shape=[act_cm, st_f32],
        compiler_params=par,
    )(y1, st1, g1r, be1r, w2b)

    y3, st3 = pl.pallas_call(
        functools.partial(_s3_kernel, m=M),
        grid=(nst,),
        in_specs=[cm, stat_i, row, row, mat((C, C))],
        out_specs=[cm, stat_o],
        out_shape=[act_cm, st_f32],
        compiler_params=par,
    )(y2, st2, g2r, be2r, w3b)

    out = pl.pallas_call(
        functools.partial(_s4_kernel, m=M, hw=HW),
        grid=(nst,),
        in_specs=[cm, stat_i, row, row, cm],
        out_specs=img,
        out_shape=jax.ShapeDtypeStruct((N, C, HW), bf16),
        compiler_params=par,
    )(y3, st3, g3r, be3r, xf)

    return out.reshape(N, C, H, W).astype(f32)


# speedup vs baseline: 1.1004x; 1.0023x over previous
import functools

import jax
import jax.numpy as jnp
from jax.experimental import pallas as pl
from jax.experimental.pallas import tpu as pltpu

_EPS = 1e-5
_CONTRACT0 = (((0,), (0,)), ((), ()))


def _lshift(a, s):
    if s == 0:
        return a
    pad = jnp.zeros((a.shape[0], abs(s)), a.dtype)
    if s > 0:
        return jnp.concatenate([a[:, s:], pad], axis=1)
    return jnp.concatenate([pad, a[:, :s]], axis=1)


def _stats(y):
    s = jnp.sum(y, axis=1, keepdims=True)
    ss = jnp.sum(y * y, axis=1, keepdims=True)
    return jnp.concatenate([s, ss], axis=1)[None]


def _fold(st_ref, g_ref, be_ref, m):
    tot = jnp.sum(st_ref[...], axis=0)
    mean = tot[:, 0:1] / m
    var = tot[:, 1:2] / m - mean * mean
    sc = jnp.transpose(g_ref[...]) * jax.lax.rsqrt(var + _EPS)
    sh = jnp.transpose(be_ref[...]) - mean * sc
    return sc, sh


def _s1_kernel(x_ref, w_ref, y_ref, xf_ref, st_ref):
    g = x_ref.shape[0]
    xc = jnp.concatenate([x_ref[i].astype(jnp.bfloat16) for i in range(g)],
                         axis=1)
    xf_ref[...] = xc
    y = jax.lax.dot_general(w_ref[...], xc, _CONTRACT0,
                            preferred_element_type=jnp.float32)
    y_ref[...] = y.astype(jnp.bfloat16)
    st_ref[...] = _stats(y)


def _s2_kernel(y1_ref, st_ref, g_ref, be_ref, w_ref, y_ref, st2_ref, *, m, width, hw):
    sc, sh = _fold(st_ref, g_ref, be_ref, m)
    c, tl = y1_ref.shape
    lane = jax.lax.broadcasted_iota(jnp.int32, (1, tl), 1)
    col = lane % width
    rowpos = lane % hw
    scb, shb = sc.astype(jnp.bfloat16), sh.astype(jnp.bfloat16)
    zero = jnp.bfloat16(0)
    a = jnp.maximum(y1_ref[...] * scb + shb, zero)
    a_l = jnp.where(col > 0, _lshift(a, -1), zero)
    a_r = jnp.where(col < width - 1, _lshift(a, 1), zero)
    a3 = jnp.concatenate([a_l, a, a_r], axis=0)
    k3 = 3 * c
    t_up = jnp.where(rowpos >= width, _lshift(a3, -width), zero)
    y = jax.lax.dot_general(w_ref[0:k3], t_up, _CONTRACT0,
                            preferred_element_type=jnp.float32)
    y = y + jax.lax.dot_general(w_ref[k3:2 * k3], a3, _CONTRACT0,
                                preferred_element_type=jnp.float32)
    t_dn = jnp.where(rowpos < hw - width, _lshift(a3, width), zero)
    y = y + jax.lax.dot_general(w_ref[2 * k3:3 * k3], t_dn, _CONTRACT0,
                                preferred_element_type=jnp.float32)
    y_ref[...] = y.astype(jnp.bfloat16)
    st2_ref[...] = _stats(y)


def _s3_kernel(y2_ref, st_ref, g_ref, be_ref, w_ref, y_ref, st3_ref, *, m):
    sc, sh = _fold(st_ref, g_ref, be_ref, m)
    scb, shb = sc.astype(jnp.bfloat16), sh.astype(jnp.bfloat16)
    a = jnp.maximum(y2_ref[...] * scb + shb, jnp.bfloat16(0))
    y = jax.lax.dot_general(w_ref[...], a, _CONTRACT0,
                            preferred_element_type=jnp.float32)
    y_ref[...] = y.astype(jnp.bfloat16)
    st3_ref[...] = _stats(y)


def _s4_kernel(y3_ref, st_ref, g_ref, be_ref, x_ref, o_ref, *, m, hw):
    sc, sh = _fold(st_ref, g_ref, be_ref, m)
    o = jnp.maximum(y3_ref[...].astype(jnp.float32) * sc + sh + x_ref[...], 0.0)
    o = o.astype(jnp.bfloat16)
    for i in range(o_ref.shape[0]):
        o_ref[i] = o[:, i * hw:(i + 1) * hw]


def kernel(w1, b1, g1, be1, w2, b2, g2, be2, w3, b3, g3, be3, x):
    N, C, H, W = x.shape
    HW = H * W
    M = N * HW
    f32, bf16 = jnp.float32, jnp.bfloat16
    G = N
    for g in range(1, N + 1):
        if N % g == 0 and (g * HW) % 128 == 0:
            G = g
            break
    nst = N // G
    TL = G * HW

    x3 = x.reshape(N, C, HW)
    w1b = w1.astype(bf16)
    w2b = w2.reshape(9 * C, C).astype(bf16)
    w3b = w3.astype(bf16)
    g1r, be1r = g1.reshape(1, C), be1.reshape(1, C)
    g2r, be2r = g2.reshape(1, C), be2.reshape(1, C)
    g3r, be3r = g3.reshape(1, C), be3.reshape(1, C)

    par = pltpu.CompilerParams(dimension_semantics=("parallel",))
    img = pl.BlockSpec((G, C, HW), lambda n: (n, 0, 0))
    cm = pl.BlockSpec((C, TL), lambda n: (0, n))
    stat_o = pl.BlockSpec((1, C, 2), lambda n: (n, 0, 0))
    stat_i = pl.BlockSpec((nst, C, 2), lambda n: (0, 0, 0))
    row = pl.BlockSpec((1, C), lambda n: (0, 0))

    def mat(shape):
        return pl.BlockSpec(shape, lambda n: (0, 0))

    act_cm = jax.ShapeDtypeStruct((C, M), bf16)
    st_f32 = jax.ShapeDtypeStruct((nst, C, 2), f32)

    y1, xf, st1 = pl.pallas_call(
        _s1_kernel,
        grid=(nst,),
        in_specs=[img, mat((C, C))],
        out_specs=[cm, cm, stat_o],
        out_shape=[act_cm, act_cm, st_f32],
        compiler_params=par,
    )(x3, w1b)

    y2, st2 = pl.pallas_call(
        functools.partial(_s2_kernel, m=M, width=W, hw=HW),
        grid=(nst,),
        in_specs=[cm, stat_i, row, row, mat((9 * C, C))],
        out_specs=[cm, stat_o],
        out_shape=[act_cm, st_f32],
        compiler_params=par,
    )(y1, st1, g1r, be1r, w2b)

    y3, st3 = pl.pallas_call(
        functools.partial(_s3_kernel, m=M),
        grid=(nst,),
        in_specs=[cm, stat_i, row, row, mat((C, C))],
        out_specs=[cm, stat_o],
        out_shape=[act_cm, st_f32],
        compiler_params=par,
    )(y2, st2, g2r, be2r, w3b)

    out = pl.pallas_call(
        functools.partial(_s4_kernel, m=M, hw=HW),
        grid=(nst,),
        in_specs=[cm, stat_i, row, row, cm],
        out_specs=img,
        out_shape=jax.ShapeDtypeStruct((N, C, HW), bf16),
        compiler_params=par,
    )(y3, st3, g3r, be3r, xf)

    return out.reshape(N, C, H, W).astype(f32)
